# baseline (speedup 1.0000x reference)
import jax
import jax.numpy as jnp
from jax.experimental import pallas as pl
from jax.experimental.pallas import tpu as pltpu


def kernel(tokens, W_pos):
    batch, seq = tokens.shape
    d = W_pos.shape[1]
    blk = 1024
    nblk = seq // blk

    def body(w_ref, o_ref, sem):
        j = pl.program_id(0)
        copies = [
            pltpu.make_async_copy(
                w_ref,
                o_ref.at[b, pl.ds(j * blk, blk), :],
                sem.at[b],
            )
            for b in range(batch)
        ]
        for c in copies:
            c.start()
        for c in copies:
            c.wait()

    out = pl.pallas_call(
        body,
        grid=(nblk,),
        in_specs=[pl.BlockSpec((blk, d), lambda j: (j, 0))],
        out_specs=pl.BlockSpec(memory_space=pl.ANY),
        out_shape=jax.ShapeDtypeStruct((batch, seq, d), W_pos.dtype),
        scratch_shapes=[pltpu.SemaphoreType.DMA((batch,))],
    )(W_pos)
    return out
